# baseline (speedup 1.0000x reference)
import jax
import jax.numpy as jnp
from jax.experimental import pallas as pl

_SCORE_T = 0.3
_IOU_T = 0.5
_K = 300
_KPAD = 384
_NEG = -1e30


def _maxarg_body(x_ref, m_ref, a_ref):
    x = x_ref[0]
    m = jnp.max(x, axis=1)
    iota_c = jax.lax.broadcasted_iota(jnp.int32, x.shape, 1)
    a = jnp.min(jnp.where(x == m[:, None], iota_c, jnp.int32(1 << 20)), axis=1)
    m_ref[0, 0, :] = m
    a_ref[0, 0, :] = a + 1


def _nms_body(s_ref, i_ref, reg_ref, anc_ref, out_ref):
    sl = s_ref[...]
    gif = i_ref[...].astype(jnp.float32)
    s = 1.0 / (1.0 + jnp.exp(-sl))

    dx = reg_ref[:, 0, :] * 0.1
    dy = reg_ref[:, 1, :] * 0.1
    dw = reg_ref[:, 2, :] * 0.2
    dh = reg_ref[:, 3, :] * 0.2
    a0 = anc_ref[:, 0, :]
    a1 = anc_ref[:, 1, :]
    a2 = anc_ref[:, 2, :]
    a3 = anc_ref[:, 3, :]
    aw = a2 - a0
    ah = a3 - a1
    acx = a0 + aw * 0.5
    acy = a1 + ah * 0.5
    pcx = dx * aw + acx
    pcy = dy * ah + acy
    pw = jnp.exp(dw) * aw
    ph = jnp.exp(dh) * ah
    x1 = pcx - pw * 0.5
    y1 = pcy - ph * 0.5
    x2 = pcx + pw * 0.5
    y2 = pcy + ph * 0.5

    m = s >= _SCORE_T
    mc = jnp.maximum(jnp.maximum(x1, y1), jnp.maximum(x2, y2))
    maxc = jnp.max(jnp.where(m, mc, _NEG), axis=1, keepdims=True)
    maxc = jnp.where(maxc > -1e29, maxc, 0.0)
    off = gif * (maxc + 1.0)
    x1o = x1 + off
    y1o = y1 + off
    x2o = x2 + off
    y2o = y2 + off
    areas = (x2o - x1o + 1.0) * (y2o - y1o + 1.0)

    ar = jax.lax.broadcasted_iota(jnp.int32, sl.shape, 1)
    sf32 = jnp.float32

    def body(i, supp):
        ohf = (ar == i).astype(sf32)

        def ext(v):
            return jnp.sum(v * ohf, axis=1, keepdims=True)

        keep_i = 1.0 - ext(supp)
        xi = ext(x1o)
        yi = ext(y1o)
        xxi = ext(x2o)
        yyi = ext(y2o)
        ai = ext(areas)
        xmin = jnp.maximum(x1o, xi)
        ymin = jnp.maximum(y1o, yi)
        xmax = jnp.minimum(x2o, xxi)
        ymax = jnp.minimum(y2o, yyi)
        inter = jnp.maximum(xmax - xmin, 0.0) * jnp.maximum(ymax - ymin, 0.0)
        iou = inter / (ai + areas - inter + 1e-16)
        new = ((keep_i > 0.5) & (iou > _IOU_T) & (ar > i)).astype(sf32)
        return jnp.maximum(supp, new)

    supp = jax.lax.fori_loop(0, _K, body, 1.0 - m.astype(sf32))
    keep = supp < 0.5
    kf = keep.astype(sf32)

    j0 = jax.lax.broadcasted_iota(jnp.int32, (_KPAD, _KPAD), 0)
    j1 = jax.lax.broadcasted_iota(jnp.int32, (_KPAD, _KPAD), 1)
    ut = (j0 <= j1).astype(sf32)
    pos = jnp.dot(kf, ut, preferred_element_type=sf32)
    dest = jnp.where(keep, pos - 1.0, 1e6).astype(jnp.int32)

    rows = jax.lax.broadcasted_iota(jnp.int32, (_KPAD, _KPAD), 0)
    ci = jax.lax.broadcasted_iota(jnp.int32, (_KPAD, 8), 1)
    for b in range(8):
        db = dest[b]
        mat = jnp.where(rows == db[None, :], 1.0, 0.0)
        cols = (s[b], gif[b], x1[b], y1[b], x2[b], y2[b])
        v = jnp.zeros((_KPAD, 8), sf32)
        for c, col in enumerate(cols):
            v = jnp.where(ci == c, col[:, None], v)
        out_ref[b, :, :] = jnp.dot(mat, v, preferred_element_type=sf32)


def kernel(cls_logits, reg_preds, anchors):
    B, N, C = cls_logits.shape
    blk = 2000 if N % 2000 == 0 else N
    nblk = N // blk

    maxlog, clsind = pl.pallas_call(
        _maxarg_body,
        grid=(B * nblk,),
        in_specs=[
            pl.BlockSpec((1, blk, C), lambda g, _nb=nblk: (g // _nb, g % _nb, 0)),
        ],
        out_specs=[
            pl.BlockSpec((1, 1, blk), lambda g: (g, 0, 0)),
            pl.BlockSpec((1, 1, blk), lambda g: (g, 0, 0)),
        ],
        out_shape=[
            jax.ShapeDtypeStruct((B * nblk, 1, blk), jnp.float32),
            jax.ShapeDtypeStruct((B * nblk, 1, blk), jnp.int32),
        ],
    )(cls_logits)
    maxlog = maxlog.reshape(B, N)
    clsind = clsind.reshape(B, N)

    top_l, top_i = jax.lax.top_k(maxlog, _K)
    g_idx = jnp.take_along_axis(clsind, top_i, axis=1)
    g_reg = jnp.take_along_axis(reg_preds, top_i[..., None], axis=1)
    g_anc = anchors[top_i]

    pad = _KPAD - _K
    top_l = jnp.pad(top_l, ((0, 0), (0, pad)), constant_values=-1e4)
    g_idx = jnp.pad(g_idx, ((0, 0), (0, pad)))
    g_reg = jnp.pad(g_reg, ((0, 0), (0, pad), (0, 0))).transpose(0, 2, 1)
    g_anc = jnp.pad(g_anc, ((0, 0), (0, pad), (0, 0))).transpose(0, 2, 1)

    out = pl.pallas_call(
        _nms_body,
        out_shape=jax.ShapeDtypeStruct((B, _KPAD, 8), jnp.float32),
    )(top_l, g_idx, g_reg, g_anc)

    out_s = out[:, :_K, 0]
    out_i = out[:, :_K, 1].astype(jnp.int32)
    out_b = out[:, :_K, 2:6]
    return (out_s, out_i, out_b)
